# baseline (speedup 1.0000x reference)
import jax
import jax.numpy as jnp
from jax import lax
from jax.experimental import pallas as pl
from jax.experimental.pallas import tpu as pltpu
from jax.experimental.pallas import tpu_sc as plsc

B = 16
N = 2048
D = 1024
NC = 2
NS = 16
BL = B // NC
RB = 48
NV = D // 16
FB = 8
TBR = 128
NTB = N // TBR
TC_NUM, TC_DEN = 3, 4
NEG = -99999.0
BIG = 1 << 30


def _sc_body(hidden, counts, tcnb, pv, pi,
             counts_v, tcnb_v, buf0, buf1, acc_v, acc_i, sem0, sem1):
    c = lax.axis_index("c")
    s = lax.axis_index("s")

    pltpu.sync_copy(counts, counts_v)
    pltpu.sync_copy(tcnb, tcnb_v)
    lane = lax.broadcasted_iota(jnp.int32, (16,), 0)
    eff_all = jnp.minimum(jnp.maximum(counts_v[...], 1), N)
    sc0_all = tcnb_v[...] * TBR

    negv = jnp.full((16,), NEG, jnp.float32)
    zerov = jnp.zeros((16,), jnp.int32)

    def init_b(bl, _):
        for v in range(NV):
            acc_v[pl.ds(bl * D + v * 16, 16)] = negv
            acc_i[pl.ds(bl * D + v * 16, 16)] = zerov
        return 0
    lax.fori_loop(0, BL, init_b, 0)

    effs, sc0s, nchs = [], [], []
    for t in range(BL):
        sel = lane == c * BL + t
        e = jnp.max(jnp.where(sel, eff_all, 0))
        s0 = jnp.max(jnp.where(sel, sc0_all, 0))
        effs.append(e)
        sc0s.append(s0)
        nchs.append((e - s0 + (RB - 1)) // RB)
    pref = [jnp.int32(0)]
    for t in range(BL):
        pref.append(pref[t] + nchs[t])
    total = pref[BL]
    mine = jnp.maximum((total - s + (NS - 1)) // NS, 0)

    def locate(j):
        b_l = jnp.int32(0)
        base = jnp.int32(0)
        e_s = effs[0]
        s0 = sc0s[0]
        for t in range(1, BL):
            cond = j >= pref[t]
            b_l = jnp.where(cond, t, b_l)
            base = jnp.where(cond, pref[t], base)
            e_s = jnp.where(cond, effs[t], e_s)
            s0 = jnp.where(cond, sc0s[t], s0)
        return b_l, j - base, e_s, s0

    def dma(b_l, k, s0, buf, sem):
        start = pl.multiple_of(jnp.minimum(s0 + k * RB, N - RB), 8)
        return pltpu.make_async_copy(
            hidden.at[c * BL + b_l, pl.ds(start, RB), :], buf, sem)

    def compute_item(b_l, k, e_s, s0, buf):
        a = s0 + k * RB
        delta = a - jnp.minimum(a, N - RB)
        cnt = jnp.minimum(RB, e_s - a)
        npair = cnt // 2
        for fb in range(FB):
            m = [acc_v[pl.ds(b_l * D + fb * 128 + v * 16, 16)]
                 for v in range(8)]
            ii = [acc_i[pl.ds(b_l * D + fb * 128 + v * 16, 16)]
                  for v in range(8)]
            g = jnp.full((16,), a, jnp.int32)

            def pair_body(r2, carry):
                mm = list(carry[0:8])
                jj = list(carry[8:16])
                gg = carry[16]
                gg1 = gg + 1
                r = delta + 2 * r2
                for v in range(8):
                    xa = buf[r, pl.ds(fb * 128 + v * 16, 16)]
                    xb = buf[r + 1, pl.ds(fb * 128 + v * 16, 16)]
                    pm = jnp.maximum(xa, xb)
                    pidx = jnp.where(xb > xa, gg1, gg)
                    cge = pm > mm[v]
                    mm[v] = jnp.maximum(mm[v], pm)
                    jj[v] = jnp.where(cge, pidx, jj[v])
                return tuple(mm) + tuple(jj) + (gg + 2,)

            out = lax.fori_loop(0, npair, pair_body,
                                tuple(m) + tuple(ii) + (g,))
            for v in range(8):
                acc_v[pl.ds(b_l * D + fb * 128 + v * 16, 16)] = out[v]
                acc_i[pl.ds(b_l * D + fb * 128 + v * 16, 16)] = out[8 + v]

        @pl.when(cnt % 2 == 1)
        def _():
            gt = jnp.full((16,), a + cnt - 1, jnp.int32)
            for v in range(NV):
                x = buf[delta + cnt - 1, pl.ds(v * 16, 16)]
                mv = acc_v[pl.ds(b_l * D + v * 16, 16)]
                iv = acc_i[pl.ds(b_l * D + v * 16, 16)]
                cge = x > mv
                acc_v[pl.ds(b_l * D + v * 16, 16)] = jnp.where(cge, x, mv)
                acc_i[pl.ds(b_l * D + v * 16, 16)] = jnp.where(cge, gt, iv)

    b0, k0, e0, s00 = locate(s)

    @pl.when(mine > 0)
    def _():
        dma(b0, k0, s00, buf0, sem0).start()

    bufs = (buf0, buf1)
    sems = (sem0, sem1)

    def pair_loop(p, _):
        for q in (0, 1):
            item = 2 * p + q

            @pl.when(item < mine)
            def _(item=item, q=q):
                j = s + NS * item
                b_l, k, e_s, s0 = locate(j)
                dma(b_l, k, s0, bufs[q], sems[q]).wait()
                nitem = item + 1

                @pl.when(nitem < mine)
                def _():
                    nb, nk, _ne, ns0 = locate(s + NS * nitem)
                    dma(nb, nk, ns0, bufs[1 - q], sems[1 - q]).start()

                compute_item(b_l, k, e_s, s0, bufs[q])
        return 0

    lax.fori_loop(0, (mine + 1) // 2, pair_loop, 0)

    w = c * NS + s
    for bl in range(BL):
        pltpu.sync_copy(acc_v.at[pl.ds(bl * D, D)],
                        pv.at[pl.ds((w * BL + bl) * D, D)])
        pltpu.sync_copy(acc_i.at[pl.ds(bl * D, D)],
                        pi.at[pl.ds((w * BL + bl) * D, D)])


_mesh = plsc.VectorSubcoreMesh(core_axis_name="c", subcore_axis_name="s")

_sc_call = pl.kernel(
    _sc_body,
    out_type=(
        jax.ShapeDtypeStruct((NC * NS * BL * D,), jnp.float32),
        jax.ShapeDtypeStruct((NC * NS * BL * D,), jnp.int32),
    ),
    mesh=_mesh,
    scratch_types=[
        pltpu.VMEM((16,), jnp.int32),
        pltpu.VMEM((16,), jnp.int32),
        pltpu.VMEM((RB, D), jnp.float32),
        pltpu.VMEM((RB, D), jnp.float32),
        pltpu.VMEM((BL * D,), jnp.float32),
        pltpu.VMEM((BL * D,), jnp.int32),
        pltpu.SemaphoreType.DMA,
        pltpu.SemaphoreType.DMA,
    ],
    compiler_params=pltpu.CompilerParams(needs_layout_passes=False),
)


def _tc_ragged(tcnb_ref, hid_ref, outv_ref, outi_ref,
               accv, acci, buf0, buf1, sem0, sem1):
    accv[...] = jnp.full((B * D,), NEG, jnp.float32)
    acci[...] = jnp.zeros((B * D,), jnp.int32)

    pref = [0]
    for t in range(B):
        pref.append(pref[t] + tcnb_ref[t])
    total = pref[B]

    def locate(j):
        bb = jnp.int32(0)
        base = jnp.int32(0)
        for t in range(1, B):
            cond = j >= pref[t]
            bb = jnp.where(cond, t, bb)
            base = jnp.where(cond, pref[t], base)
        return bb, j - base

    def dma(bb, k, buf, sem):
        return pltpu.make_async_copy(
            hid_ref.at[bb, pl.ds(k * TBR, TBR), :], buf, sem)

    def compute(bb, k, buf):
        x = buf[...]
        rid = lax.broadcasted_iota(jnp.int32, (TBR, D), 0) + k * TBR
        mx = jnp.max(x, axis=0)
        amn = jnp.min(jnp.where(x == mx[None, :], rid, BIG), axis=0)
        off = bb * D
        m = accv[pl.ds(off, D)]
        take = mx > m
        accv[pl.ds(off, D)] = jnp.where(take, mx, m)
        ii = acci[pl.ds(off, D)]
        acci[pl.ds(off, D)] = jnp.where(take, amn, ii)

    b0, k0 = locate(0)

    @pl.when(total > 0)
    def _():
        dma(b0, k0, buf0, sem0).start()

    bufs = (buf0, buf1)
    sems = (sem0, sem1)

    def pair_loop(p, _):
        for q in (0, 1):
            item = 2 * p + q

            @pl.when(item < total)
            def _(item=item, q=q):
                bb, k = locate(item)
                dma(bb, k, bufs[q], sems[q]).wait()

                @pl.when(item + 1 < total)
                def _():
                    nb, nk = locate(item + 1)
                    dma(nb, nk, bufs[1 - q], sems[1 - q]).start()

                compute(bb, k, bufs[q])
        return 0

    lax.fori_loop(0, (total + 1) // 2, pair_loop, 0)

    outv_ref[...] = accv[...]
    outi_ref[...] = acci[...]


_tc_ragged_call = pl.pallas_call(
    _tc_ragged,
    in_specs=[
        pl.BlockSpec(memory_space=pltpu.SMEM),
        pl.BlockSpec(memory_space=pltpu.HBM),
    ],
    out_shape=(
        jax.ShapeDtypeStruct((B * D,), jnp.float32),
        jax.ShapeDtypeStruct((B * D,), jnp.int32),
    ),
    scratch_shapes=[
        pltpu.VMEM((B * D,), jnp.float32),
        pltpu.VMEM((B * D,), jnp.int32),
        pltpu.VMEM((TBR, D), jnp.float32),
        pltpu.VMEM((TBR, D), jnp.float32),
        pltpu.SemaphoreType.DMA,
        pltpu.SemaphoreType.DMA,
    ],
)


def _tc_combine(tcv_ref, tci_ref, pv_ref, pi_ref, outv_ref, outi_ref):
    col = lax.broadcasted_iota(jnp.int32, (D,), 0)
    for bg in range(B):
        c, bl = bg // BL, bg % BL
        m = tcv_ref[pl.ds(bg * D, D)]
        ii = tci_ref[pl.ds(bg * D, D)]
        for j in range(NS):
            off = ((c * NS + j) * BL + bl) * D
            x = pv_ref[pl.ds(off, D)]
            ix = pi_ref[pl.ds(off, D)]
            take = (x > m) | ((x == m) & (ix < ii))
            m = jnp.where(take, x, m)
            ii = jnp.where(take, ix, ii)
        outv_ref[bg, 0, :] = m
        outi_ref[bg, 0, 0, :] = ii * D + col


_tc_combine_call = pl.pallas_call(
    _tc_combine,
    out_shape=(
        jax.ShapeDtypeStruct((B, 1, D), jnp.float32),
        jax.ShapeDtypeStruct((B, 1, 1, D), jnp.int32),
    ),
)


@jax.jit
def kernel(hidden, child_counts):
    eff = jnp.minimum(jnp.maximum(child_counts, 1), N)
    tc_nb = (eff * TC_NUM) // (TC_DEN * TBR)
    pv, pi = _sc_call(hidden, child_counts, tc_nb)
    tcv, tci = _tc_ragged_call(tc_nb, hidden)
    return _tc_combine_call(tcv, tci, pv, pi)


# speedup vs baseline: 1.6654x; 1.6654x over previous
import jax
import jax.numpy as jnp
from jax import lax
from jax.experimental import pallas as pl
from jax.experimental.pallas import tpu as pltpu
from jax.experimental.pallas import tpu_sc as plsc

B = 16
N = 2048
D = 1024
NC = 2
NS = 16
BL = B // NC
RB = 48
NV = D // 16
FB = 8
TBR = 256
NTB = N // TBR
TC_NUM, TC_DEN = 3, 4
NEG = -99999.0
BIG = 1 << 30


def _sc_body(hidden, counts, tcnb, pv, pi,
             counts_v, tcnb_v, buf0, buf1, acc_v, acc_i, sem0, sem1):
    c = lax.axis_index("c")
    s = lax.axis_index("s")

    pltpu.sync_copy(counts, counts_v)
    pltpu.sync_copy(tcnb, tcnb_v)
    lane = lax.broadcasted_iota(jnp.int32, (16,), 0)
    eff_all = jnp.minimum(jnp.maximum(counts_v[...], 1), N)
    sc0_all = tcnb_v[...] * TBR

    negv = jnp.full((16,), NEG, jnp.float32)
    zerov = jnp.zeros((16,), jnp.int32)

    def init_b(bl, _):
        for v in range(NV):
            acc_v[pl.ds(bl * D + v * 16, 16)] = negv
            acc_i[pl.ds(bl * D + v * 16, 16)] = zerov
        return 0
    lax.fori_loop(0, BL, init_b, 0)

    effs, sc0s, nchs = [], [], []
    for t in range(BL):
        sel = lane == c * BL + t
        e = jnp.max(jnp.where(sel, eff_all, 0))
        s0 = jnp.max(jnp.where(sel, sc0_all, 0))
        effs.append(e)
        sc0s.append(s0)
        nchs.append((e - s0 + (RB - 1)) // RB)
    pref = [jnp.int32(0)]
    for t in range(BL):
        pref.append(pref[t] + nchs[t])
    total = pref[BL]
    mine = jnp.maximum((total - s + (NS - 1)) // NS, 0)

    def locate(j):
        b_l = jnp.int32(0)
        base = jnp.int32(0)
        e_s = effs[0]
        s0 = sc0s[0]
        for t in range(1, BL):
            cond = j >= pref[t]
            b_l = jnp.where(cond, t, b_l)
            base = jnp.where(cond, pref[t], base)
            e_s = jnp.where(cond, effs[t], e_s)
            s0 = jnp.where(cond, sc0s[t], s0)
        return b_l, j - base, e_s, s0

    def dma(b_l, k, s0, buf, sem):
        start = pl.multiple_of(jnp.minimum(s0 + k * RB, N - RB), 8)
        return pltpu.make_async_copy(
            hidden.at[c * BL + b_l, pl.ds(start, RB), :], buf, sem)

    def compute_item(b_l, k, e_s, s0, buf):
        a = s0 + k * RB
        delta = a - jnp.minimum(a, N - RB)
        cnt = jnp.minimum(RB, e_s - a)
        npair = cnt // 2
        for fb in range(FB):
            m = [acc_v[pl.ds(b_l * D + fb * 128 + v * 16, 16)]
                 for v in range(8)]
            ii = [acc_i[pl.ds(b_l * D + fb * 128 + v * 16, 16)]
                  for v in range(8)]
            g = jnp.full((16,), a, jnp.int32)

            def pair_body(r2, carry):
                mm = list(carry[0:8])
                jj = list(carry[8:16])
                gg = carry[16]
                gg1 = gg + 1
                r = delta + 2 * r2
                for v in range(8):
                    xa = buf[r, pl.ds(fb * 128 + v * 16, 16)]
                    xb = buf[r + 1, pl.ds(fb * 128 + v * 16, 16)]
                    pm = jnp.maximum(xa, xb)
                    pidx = jnp.where(xb > xa, gg1, gg)
                    cge = pm > mm[v]
                    mm[v] = jnp.maximum(mm[v], pm)
                    jj[v] = jnp.where(cge, pidx, jj[v])
                return tuple(mm) + tuple(jj) + (gg + 2,)

            out = lax.fori_loop(0, npair, pair_body,
                                tuple(m) + tuple(ii) + (g,))
            for v in range(8):
                acc_v[pl.ds(b_l * D + fb * 128 + v * 16, 16)] = out[v]
                acc_i[pl.ds(b_l * D + fb * 128 + v * 16, 16)] = out[8 + v]

        @pl.when(cnt % 2 == 1)
        def _():
            gt = jnp.full((16,), a + cnt - 1, jnp.int32)
            for v in range(NV):
                x = buf[delta + cnt - 1, pl.ds(v * 16, 16)]
                mv = acc_v[pl.ds(b_l * D + v * 16, 16)]
                iv = acc_i[pl.ds(b_l * D + v * 16, 16)]
                cge = x > mv
                acc_v[pl.ds(b_l * D + v * 16, 16)] = jnp.where(cge, x, mv)
                acc_i[pl.ds(b_l * D + v * 16, 16)] = jnp.where(cge, gt, iv)

    b0, k0, e0, s00 = locate(s)

    @pl.when(mine > 0)
    def _():
        dma(b0, k0, s00, buf0, sem0).start()

    bufs = (buf0, buf1)
    sems = (sem0, sem1)

    def pair_loop(p, _):
        for q in (0, 1):
            item = 2 * p + q

            @pl.when(item < mine)
            def _(item=item, q=q):
                j = s + NS * item
                b_l, k, e_s, s0 = locate(j)
                dma(b_l, k, s0, bufs[q], sems[q]).wait()
                nitem = item + 1

                @pl.when(nitem < mine)
                def _():
                    nb, nk, _ne, ns0 = locate(s + NS * nitem)
                    dma(nb, nk, ns0, bufs[1 - q], sems[1 - q]).start()

                compute_item(b_l, k, e_s, s0, bufs[q])
        return 0

    lax.fori_loop(0, (mine + 1) // 2, pair_loop, 0)

    w = c * NS + s
    for bl in range(BL):
        pltpu.sync_copy(acc_v.at[pl.ds(bl * D, D)],
                        pv.at[pl.ds((w * BL + bl) * D, D)])
        pltpu.sync_copy(acc_i.at[pl.ds(bl * D, D)],
                        pi.at[pl.ds((w * BL + bl) * D, D)])


_mesh = plsc.VectorSubcoreMesh(core_axis_name="c", subcore_axis_name="s")

_sc_call = pl.kernel(
    _sc_body,
    out_type=(
        jax.ShapeDtypeStruct((NC * NS * BL * D,), jnp.float32),
        jax.ShapeDtypeStruct((NC * NS * BL * D,), jnp.int32),
    ),
    mesh=_mesh,
    scratch_types=[
        pltpu.VMEM((16,), jnp.int32),
        pltpu.VMEM((16,), jnp.int32),
        pltpu.VMEM((RB, D), jnp.float32),
        pltpu.VMEM((RB, D), jnp.float32),
        pltpu.VMEM((BL * D,), jnp.float32),
        pltpu.VMEM((BL * D,), jnp.int32),
        pltpu.SemaphoreType.DMA,
        pltpu.SemaphoreType.DMA,
    ],
    compiler_params=pltpu.CompilerParams(needs_layout_passes=False),
)


def _tc_ragged(tcnb_ref, x_ref, outv_ref, outi_ref):
    j = pl.program_id(1)
    b = pl.program_id(0)
    nb = tcnb_ref[b]

    @pl.when(j == 0)
    def _():
        outv_ref[...] = jnp.full((1, 1, D), NEG, jnp.float32)
        outi_ref[...] = jnp.zeros((1, 1, D), jnp.int32)

    @pl.when(j < nb)
    def _():
        x = x_ref[0]
        rid = lax.broadcasted_iota(jnp.int32, (TBR, D), 0) + j * TBR
        mx = jnp.max(x, axis=0, keepdims=True)
        amn = jnp.min(jnp.where(x == mx, rid, BIG), axis=0, keepdims=True)
        m = outv_ref[0]
        take = mx > m
        outv_ref[0] = jnp.where(take, mx, m)
        outi_ref[0] = jnp.where(take, amn, outi_ref[0])


_tc_ragged_call = pl.pallas_call(
    _tc_ragged,
    grid_spec=pltpu.PrefetchScalarGridSpec(
        num_scalar_prefetch=1,
        grid=(B, NTB),
        in_specs=[
            pl.BlockSpec(
                (1, TBR, D),
                lambda b, j, tcnb: (b, jnp.minimum(j, jnp.maximum(tcnb[b] - 1, 0)), 0)),
        ],
        out_specs=[
            pl.BlockSpec((1, 1, D), lambda b, j, tcnb: (b, 0, 0)),
            pl.BlockSpec((1, 1, D), lambda b, j, tcnb: (b, 0, 0)),
        ],
    ),
    out_shape=(
        jax.ShapeDtypeStruct((B, 1, D), jnp.float32),
        jax.ShapeDtypeStruct((B, 1, D), jnp.int32),
    ),
)


def _tc_combine(tcv_ref, tci_ref, pv_ref, pi_ref, outv_ref, outi_ref):
    col = lax.broadcasted_iota(jnp.int32, (D,), 0)
    for bg in range(B):
        c, bl = bg // BL, bg % BL
        m = tcv_ref[bg, 0]
        ii = tci_ref[bg, 0]
        for j in range(NS):
            off = ((c * NS + j) * BL + bl) * D
            x = pv_ref[pl.ds(off, D)]
            ix = pi_ref[pl.ds(off, D)]
            take = (x > m) | ((x == m) & (ix < ii))
            m = jnp.where(take, x, m)
            ii = jnp.where(take, ix, ii)
        outv_ref[bg, 0, :] = m
        outi_ref[bg, 0, 0, :] = ii * D + col


_tc_combine_call = pl.pallas_call(
    _tc_combine,
    out_shape=(
        jax.ShapeDtypeStruct((B, 1, D), jnp.float32),
        jax.ShapeDtypeStruct((B, 1, 1, D), jnp.int32),
    ),
)


@jax.jit
def kernel(hidden, child_counts):
    eff = jnp.minimum(jnp.maximum(child_counts, 1), N)
    tc_nb = (eff * TC_NUM) // (TC_DEN * TBR)
    pv, pi = _sc_call(hidden, child_counts, tc_nb)
    tcv, tci = _tc_ragged_call(tc_nb, hidden)
    return _tc_combine_call(tcv, tci, pv, pi)


# speedup vs baseline: 1.7726x; 1.0644x over previous
import jax
import jax.numpy as jnp
from jax import lax
from jax.experimental import pallas as pl
from jax.experimental.pallas import tpu as pltpu
from jax.experimental.pallas import tpu_sc as plsc

B = 16
N = 2048
D = 1024
NC = 2
NS = 16
BL = B // NC
RB = 48
NV = D // 16
FB = 8
NEG = -99999.0


def _sc_body(hidden, counts, pv, pi,
             counts_v, buf0, buf1, acc_v, acc_i, sem0, sem1):
    c = lax.axis_index("c")
    s = lax.axis_index("s")

    pltpu.sync_copy(counts, counts_v)
    lane = lax.broadcasted_iota(jnp.int32, (16,), 0)
    eff_all = jnp.minimum(jnp.maximum(counts_v[...], 1), N)

    negv = jnp.full((16,), NEG, jnp.float32)
    zerov = jnp.zeros((16,), jnp.int32)

    def init_b(bl, _):
        for v in range(NV):
            acc_v[pl.ds(bl * D + v * 16, 16)] = negv
            acc_i[pl.ds(bl * D + v * 16, 16)] = zerov
        return 0
    lax.fori_loop(0, BL, init_b, 0)

    effs, nchs = [], []
    for t in range(BL):
        e = jnp.max(jnp.where(lane == c * BL + t, eff_all, 0))
        effs.append(e)
        nchs.append((e + (RB - 1)) // RB)
    pref = [jnp.int32(0)]
    for t in range(BL):
        pref.append(pref[t] + nchs[t])
    total = pref[BL]
    mine = jnp.maximum((total - s + (NS - 1)) // NS, 0)

    def locate(j):
        b_l = jnp.int32(0)
        base = jnp.int32(0)
        e_s = effs[0]
        for t in range(1, BL):
            cond = j >= pref[t]
            b_l = jnp.where(cond, t, b_l)
            base = jnp.where(cond, pref[t], base)
            e_s = jnp.where(cond, effs[t], e_s)
        return b_l, j - base, e_s

    def dma(b_l, k, buf, sem):
        return pltpu.make_async_copy(
            hidden.at[c * BL + b_l, pl.ds(k * RB, RB), :], buf, sem)

    def compute_item(b_l, k, e_s, buf):
        cnt = jnp.minimum(RB, e_s - k * RB)
        npair = cnt // 2
        for fb in range(FB):
            m = [acc_v[pl.ds(b_l * D + fb * 128 + v * 16, 16)]
                 for v in range(8)]
            ii = [acc_i[pl.ds(b_l * D + fb * 128 + v * 16, 16)]
                  for v in range(8)]
            g = jnp.full((16,), k * RB, jnp.int32)

            def pair_body(r2, carry):
                mm = list(carry[0:8])
                jj = list(carry[8:16])
                gg = carry[16]
                gg1 = gg + 1
                r = 2 * r2
                for v in range(8):
                    xa = buf[r, pl.ds(fb * 128 + v * 16, 16)]
                    xb = buf[r + 1, pl.ds(fb * 128 + v * 16, 16)]
                    pm = jnp.maximum(xa, xb)
                    pidx = jnp.where(xb > xa, gg1, gg)
                    cge = pm > mm[v]
                    mm[v] = jnp.maximum(mm[v], pm)
                    jj[v] = jnp.where(cge, pidx, jj[v])
                return tuple(mm) + tuple(jj) + (gg + 2,)

            out = lax.fori_loop(0, npair, pair_body,
                                tuple(m) + tuple(ii) + (g,))
            for v in range(8):
                acc_v[pl.ds(b_l * D + fb * 128 + v * 16, 16)] = out[v]
                acc_i[pl.ds(b_l * D + fb * 128 + v * 16, 16)] = out[8 + v]

        @pl.when(cnt % 2 == 1)
        def _():
            gt = jnp.full((16,), k * RB + cnt - 1, jnp.int32)
            for v in range(NV):
                x = buf[cnt - 1, pl.ds(v * 16, 16)]
                mv = acc_v[pl.ds(b_l * D + v * 16, 16)]
                iv = acc_i[pl.ds(b_l * D + v * 16, 16)]
                cge = x > mv
                acc_v[pl.ds(b_l * D + v * 16, 16)] = jnp.where(cge, x, mv)
                acc_i[pl.ds(b_l * D + v * 16, 16)] = jnp.where(cge, gt, iv)

    b0, k0, e0 = locate(s)

    @pl.when(mine > 0)
    def _():
        dma(b0, k0, buf0, sem0).start()

    bufs = (buf0, buf1)
    sems = (sem0, sem1)

    def pair_loop(p, _):
        for q in (0, 1):
            item = 2 * p + q

            @pl.when(item < mine)
            def _(item=item, q=q):
                j = s + NS * item
                b_l, k, e_s = locate(j)
                dma(b_l, k, bufs[q], sems[q]).wait()
                nitem = item + 1

                @pl.when(nitem < mine)
                def _():
                    nb, nk, _ne = locate(s + NS * nitem)
                    dma(nb, nk, bufs[1 - q], sems[1 - q]).start()

                compute_item(b_l, k, e_s, bufs[q])
        return 0

    lax.fori_loop(0, (mine + 1) // 2, pair_loop, 0)

    w = c * NS + s
    for bl in range(BL):
        pltpu.sync_copy(acc_v.at[pl.ds(bl * D, D)],
                        pv.at[pl.ds((w * BL + bl) * D, D)])
        pltpu.sync_copy(acc_i.at[pl.ds(bl * D, D)],
                        pi.at[pl.ds((w * BL + bl) * D, D)])


_mesh = plsc.VectorSubcoreMesh(core_axis_name="c", subcore_axis_name="s")

_sc_call = pl.kernel(
    _sc_body,
    out_type=(
        jax.ShapeDtypeStruct((NC * NS * BL * D,), jnp.float32),
        jax.ShapeDtypeStruct((NC * NS * BL * D,), jnp.int32),
    ),
    mesh=_mesh,
    scratch_types=[
        pltpu.VMEM((16,), jnp.int32),
        pltpu.VMEM((RB, D), jnp.float32),
        pltpu.VMEM((RB, D), jnp.float32),
        pltpu.VMEM((BL * D,), jnp.float32),
        pltpu.VMEM((BL * D,), jnp.int32),
        pltpu.SemaphoreType.DMA,
        pltpu.SemaphoreType.DMA,
    ],
    compiler_params=pltpu.CompilerParams(needs_layout_passes=False),
)


def _tc_combine(pv_ref, pi_ref, outv_ref, outi_ref):
    col = lax.broadcasted_iota(jnp.int32, (D,), 0)
    for bg in range(B):
        c, bl = bg // BL, bg % BL
        m = pv_ref[pl.ds((c * NS * BL + bl) * D, D)]
        ii = pi_ref[pl.ds((c * NS * BL + bl) * D, D)]
        for j in range(1, NS):
            off = ((c * NS + j) * BL + bl) * D
            x = pv_ref[pl.ds(off, D)]
            ix = pi_ref[pl.ds(off, D)]
            take = (x > m) | ((x == m) & (ix < ii))
            m = jnp.where(take, x, m)
            ii = jnp.where(take, ix, ii)
        outv_ref[bg, 0, :] = m
        outi_ref[bg, 0, 0, :] = ii * D + col


_tc_combine_call = pl.pallas_call(
    _tc_combine,
    out_shape=(
        jax.ShapeDtypeStruct((B, 1, D), jnp.float32),
        jax.ShapeDtypeStruct((B, 1, 1, D), jnp.int32),
    ),
)


@jax.jit
def kernel(hidden, child_counts):
    pv, pi = _sc_call(hidden, child_counts)
    return _tc_combine_call(pv, pi)
